# baseline (speedup 1.0000x reference)
import functools

import jax
import jax.numpy as jnp
from jax import lax
from jax.experimental import pallas as pl
from jax.experimental.pallas import tpu as pltpu
from jax.experimental.pallas import tpu_sc as plsc

B, K, A, D = 128, 20, 64, 512
TOPK = K
BB = 64

_NEG = -3.0e38
_NW = 32
_RPW = B // _NW
_NCH = (K * A) // 16
_NG = _NCH // 16


def _tc_scores_body(z_ref, achs_ref, ret_ref, wqd_ref, wk_ref, bd_ref,
                    y_ref, disp_ref):
    zb = z_ref[...]
    ab = achs_ref[...]
    wqd = wqd_ref[...]
    wk = wk_ref[...]
    bd = bd_ref[...]

    qd = jnp.dot(ab.reshape(BB * A, D), wqd,
                 preferred_element_type=jnp.float32)
    q = qd[:, :D].reshape(BB, A, D)
    disp_ref[...] = (qd[:, D:] + bd).reshape(BB, A, D)
    kk = jnp.dot(zb.reshape(BB * K, D), wk,
                 preferred_element_type=jnp.float32).reshape(BB, K, D)

    scale = 1.0 / (D ** 0.5)
    logits = lax.dot_general(kk, q, (((2,), (2,)), ((0,), (0,))),
                             preferred_element_type=jnp.float32) * scale

    m = jnp.max(logits, axis=1, keepdims=True)
    lse = jnp.log(jnp.sum(jnp.exp(logits - m), axis=1, keepdims=True)) + m
    ret = ret_ref[...]
    y_ref[...] = logits - lse + ret[:, :, None]


def _xlane(v, op, iota):
    for s in (1, 2, 4, 8):
        v = op(v, v.at[iota ^ s].get(mode="promise_in_bounds"))
    return v


def _sc_topk_gather(y_hbm, z_hbm, d_hbm, zg_hbm, dg_hbm, val_hbm, idx_hbm,
                    ybuf, idxz, idxa, zbuf, dbuf, vbuf, fbuf, sem, sem2):
    wid = lax.axis_index("s") * 2 + lax.axis_index("c")
    iota = lax.iota(jnp.int32, 16)
    negv = jnp.full((16,), _NEG, jnp.float32)
    mask0 = iota == 0

    for r in range(_RPW):
        b = wid * _RPW + r
        roff = r * (K * A)
        pltpu.sync_copy(y_hbm.at[pl.ds(b * (K * A), K * A)],
                        ybuf.at[pl.ds(roff, K * A)])

        cm = ybuf[pl.ds(roff, 16)]
        for c in range(1, _NCH):
            cm = jnp.maximum(cm, ybuf[pl.ds(roff + 16 * c, 16)])
        val_lo = negv
        val_hi = negv
        fid_lo = jnp.zeros((16,), jnp.int32)
        fid_hi = jnp.zeros((16,), jnp.int32)
        for t in range(TOPK):
            mx = _xlane(cm, jnp.maximum, iota)
            lane = _xlane(jnp.where(cm == mx, iota, 1 << 30),
                          jnp.minimum, iota)
            fcand = jnp.full((16,), 1 << 30, jnp.int32)
            cols = []
            flats = []
            for g in range(_NG):
                flat_g = 256 * g + 16 * iota + lane
                v_g = plsc.load_gather(ybuf, [roff + flat_g])
                cols.append(v_g)
                flats.append(flat_g)
                fcand = jnp.minimum(
                    fcand, jnp.where(v_g == mx, flat_g, 1 << 30))
            fidx = _xlane(fcand, jnp.minimum, iota)
            if t < 16:
                val_lo = jnp.where(iota == t, mx, val_lo)
                fid_lo = jnp.where(iota == t, fidx, fid_lo)
            else:
                val_hi = jnp.where(iota == (t - 16), mx, val_hi)
                fid_hi = jnp.where(iota == (t - 16), fidx, fid_hi)
            if t + 1 < TOPK:
                plsc.store_scatter(ybuf, [roff + fidx], negv, mask=mask0)
                ncm = negv
                for g in range(_NG):
                    ncm = jnp.maximum(
                        ncm, jnp.where(flats[g] == fidx, _NEG, cols[g]))
                cm = jnp.where(iota == lane,
                               _xlane(ncm, jnp.maximum, iota), cm)

        vbuf[pl.ds(0, 16)] = val_lo
        vbuf[pl.ds(16, 16)] = val_hi
        fbuf[pl.ds(0, 16)] = fid_lo
        fbuf[pl.ds(16, 16)] = fid_hi
        kg_lo = b * K + lax.shift_right_logical(fid_lo, 6)
        kg_hi = b * K + lax.shift_right_logical(fid_hi, 6)
        ag_lo = b * A + (fid_lo & 63)
        ag_hi = b * A + (fid_hi & 63)
        pad = iota < (TOPK - 16)
        idxz[pl.ds(0, 16)] = kg_lo
        idxz[pl.ds(16, 16)] = jnp.where(pad, kg_hi, 0)
        idxa[pl.ds(0, 16)] = ag_lo
        idxa[pl.ds(16, 16)] = jnp.where(pad, ag_hi, 0)

        cp1 = pltpu.async_copy(z_hbm.at[idxz], zbuf, sem)
        cp2 = pltpu.async_copy(d_hbm.at[idxa], dbuf, sem2)
        cp1.wait()
        cp2.wait()
        pltpu.sync_copy(zbuf, zg_hbm.at[b])
        pltpu.sync_copy(dbuf, dg_hbm.at[b])
        pltpu.sync_copy(vbuf, val_hbm.at[pl.ds(b * 32, 32)])
        pltpu.sync_copy(fbuf, idx_hbm.at[pl.ds(b * 32, 32)])


def _tc_finish_body(zg_ref, dg_ref, val_ref, fid_ref,
                    zout_ref, att_ref, idx_ref):
    zn = (zg_ref[...] + dg_ref[...])[:, :TOPK, :]
    mu = jnp.mean(zn, axis=2, keepdims=True)
    var = jnp.mean((zn - mu) ** 2, axis=2, keepdims=True)
    zout_ref[...] = (zn - mu) / jnp.sqrt(var + 1e-5)
    v = val_ref[...]
    att_ref[...] = v[:, :TOPK] - v[:, 0:1]
    idx_ref[...] = fid_ref[...][:, :TOPK]


@jax.jit
def _run(z, achs, ret2d, Wqd, Wk, bd2d):
    grid = (B // BB,)
    y3, disp = pl.pallas_call(
        _tc_scores_body,
        grid=grid,
        in_specs=[
            pl.BlockSpec((BB, K, D), lambda i: (i, 0, 0)),
            pl.BlockSpec((BB, A, D), lambda i: (i, 0, 0)),
            pl.BlockSpec((BB, K), lambda i: (i, 0)),
            pl.BlockSpec((D, 2 * D), lambda i: (0, 0)),
            pl.BlockSpec((D, D), lambda i: (0, 0)),
            pl.BlockSpec((1, D), lambda i: (0, 0)),
        ],
        out_specs=(
            pl.BlockSpec((BB, K, A), lambda i: (i, 0, 0)),
            pl.BlockSpec((BB, A, D), lambda i: (i, 0, 0)),
        ),
        out_shape=(
            jax.ShapeDtypeStruct((B, K, A), jnp.float32),
            jax.ShapeDtypeStruct((B, A, D), jnp.float32),
        ),
    )(z, achs, ret2d, Wqd, Wk, bd2d)

    sc = functools.partial(
        pl.kernel,
        out_type=(
            jax.ShapeDtypeStruct((B, 32, D), jnp.float32),
            jax.ShapeDtypeStruct((B, 32, D), jnp.float32),
            jax.ShapeDtypeStruct((B * 32,), jnp.float32),
            jax.ShapeDtypeStruct((B * 32,), jnp.int32),
        ),
        mesh=plsc.VectorSubcoreMesh(core_axis_name="c",
                                    subcore_axis_name="s"),
        compiler_params=pltpu.CompilerParams(needs_layout_passes=False),
        scratch_types=[
            pltpu.VMEM((_RPW * K * A,), jnp.float32),
            pltpu.VMEM((32,), jnp.int32),
            pltpu.VMEM((32,), jnp.int32),
            pltpu.VMEM((32, D), jnp.float32),
            pltpu.VMEM((32, D), jnp.float32),
            pltpu.VMEM((32,), jnp.float32),
            pltpu.VMEM((32,), jnp.int32),
            pltpu.SemaphoreType.DMA,
            pltpu.SemaphoreType.DMA,
        ],
    )(_sc_topk_gather)
    zg, dg, vals, fids = sc(y3.reshape(B * K * A),
                          z.reshape(B * K, D),
                          disp.reshape(B * A, D))

    z_out, att, idx = pl.pallas_call(
        _tc_finish_body,
        grid=grid,
        in_specs=[
            pl.BlockSpec((BB, 32, D), lambda i: (i, 0, 0)),
            pl.BlockSpec((BB, 32, D), lambda i: (i, 0, 0)),
            pl.BlockSpec((BB, 32), lambda i: (i, 0)),
            pl.BlockSpec((BB, 32), lambda i: (i, 0)),
        ],
        out_specs=(
            pl.BlockSpec((BB, TOPK, D), lambda i: (i, 0, 0)),
            pl.BlockSpec((BB, TOPK), lambda i: (i, 0)),
            pl.BlockSpec((BB, TOPK), lambda i: (i, 0)),
        ),
        out_shape=(
            jax.ShapeDtypeStruct((B, TOPK, D), jnp.float32),
            jax.ShapeDtypeStruct((B, TOPK), jnp.float32),
            jax.ShapeDtypeStruct((B, TOPK), jnp.int32),
        ),
    )(zg, dg, vals.reshape(B, 32), fids.reshape(B, 32))
    return z_out, att, idx


def kernel(z, achs, anchor_att_ret, Wq, Wk, Wd, bd):
    ret2d = anchor_att_ret.reshape(B, K)
    bd2d = bd.reshape(1, D)
    Wqd = jnp.concatenate([Wq, Wd], axis=1)
    z_out, att, idx = _run(z, achs, ret2d, Wqd, Wk, bd2d)
    return (z_out, att.reshape(B, TOPK, 1), idx)


# speedup vs baseline: 1.7284x; 1.7284x over previous
import functools

import jax
import jax.numpy as jnp
from jax import lax
from jax.experimental import pallas as pl
from jax.experimental.pallas import tpu as pltpu
from jax.experimental.pallas import tpu_sc as plsc

B, K, A, D = 128, 20, 64, 512
TOPK = K
BB = 64

_NEG = -3.0e38
_NW = 32
_RPW = B // _NW
_NCH = (K * A) // 16
_NG = _NCH // 16


def _tc_scores_body(z_ref, achs_ref, ret_ref, wqd_ref, wk_ref, bd_ref,
                    y_ref, disp_ref):
    zb = z_ref[...]
    ab = achs_ref[...]
    wqd = wqd_ref[...]
    wk = wk_ref[...]
    bd = bd_ref[...]

    qd = jnp.dot(ab.reshape(BB * A, D), wqd,
                 preferred_element_type=jnp.float32)
    q = qd[:, :D].reshape(BB, A, D)
    disp_ref[...] = (qd[:, D:] + bd).reshape(BB, A, D)
    kk = jnp.dot(zb.reshape(BB * K, D), wk,
                 preferred_element_type=jnp.float32).reshape(BB, K, D)

    scale = 1.0 / (D ** 0.5)
    logits = lax.dot_general(kk, q, (((2,), (2,)), ((0,), (0,))),
                             preferred_element_type=jnp.float32) * scale

    m = jnp.max(logits, axis=1, keepdims=True)
    lse = jnp.log(jnp.sum(jnp.exp(logits - m), axis=1, keepdims=True)) + m
    ret = ret_ref[...]
    y_ref[...] = logits - lse + ret[:, :, None]


def _sc_topk_gather(y_hbm, z_hbm, d_hbm, zg_hbm, dg_hbm, val_hbm, idx_hbm,
                    ybuf, izbuf, iabuf, zbufs, dbufs, vbuf, fbuf, semg):
    wid = lax.axis_index("s") * 2 + lax.axis_index("c")
    iota = lax.iota(jnp.int32, 16)
    negv = jnp.full((16,), _NEG, jnp.float32)
    mask0 = iota == 0
    mlo4 = iota < 4
    nrow = _RPW * TOPK

    pltpu.sync_copy(y_hbm.at[pl.ds(wid * (_RPW * K * A), _RPW * K * A)],
                    ybuf)

    def _comb(a, b):
        gt = b[0] > a[0]
        return (jnp.where(gt, b[0], a[0]), jnp.where(gt, b[1], a[1]))

    def _bfly(v, i):
        for sft in (1, 2, 4, 8):
            vv = v.at[iota ^ sft].get(mode="promise_in_bounds")
            ii = i.at[iota ^ sft].get(mode="promise_in_bounds")
            gt = vv > v
            v = jnp.where(gt, vv, v)
            i = jnp.where(gt, ii, i)
        return v, i

    for r in range(_RPW):
        b = wid * _RPW + r
        base = r * (K * A)
        va = [ybuf[pl.ds(base + 16 * j, 16)] for j in range(4)]
        ia = [16 * j + iota for j in range(4)]
        for c in range(4, _NCH):
            x = ybuf[pl.ds(base + 16 * c, 16)]
            f = 16 * c + iota
            j = c & 3
            gt = x > va[j]
            va[j] = jnp.where(gt, x, va[j])
            ia[j] = jnp.where(gt, f, ia[j])
        cm, ci = _comb(_comb((va[0], ia[0]), (va[1], ia[1])),
                       _comb((va[2], ia[2]), (va[3], ia[3])))

        def _step(t, carry):
            cm, ci, v_lo, v_hi, f_lo, f_hi = carry
            mx, fidx = _bfly(cm, ci)
            v_lo = jnp.where(iota == t, mx, v_lo)
            f_lo = jnp.where(iota == t, fidx, f_lo)
            v_hi = jnp.where(iota == t - 16, mx, v_hi)
            f_hi = jnp.where(iota == t - 16, fidx, f_hi)
            plsc.store_scatter(ybuf, [base + fidx], negv, mask=mask0)
            lane = fidx & 15
            nv = None
            for g in range(_NG):
                fg = 256 * g + 16 * iota + lane
                vg = plsc.load_gather(ybuf, [base + fg])
                nv = (vg, fg) if nv is None else _comb(nv, (vg, fg))
            colv, coli = _bfly(nv[0], nv[1])
            cm = jnp.where(iota == lane, colv, cm)
            ci = jnp.where(iota == lane, coli, ci)
            return (cm, ci, v_lo, v_hi, f_lo, f_hi)

        zero = jnp.zeros((16,), jnp.int32)
        _, _, v_lo, v_hi, f_lo, f_hi = lax.fori_loop(
            0, TOPK, _step, (cm, ci, negv, negv, zero, zero))

        hi_pos = r * TOPK + 16 + jnp.minimum(iota, 3)
        plsc.store_scatter(vbuf, [r * TOPK + iota], v_lo)
        plsc.store_scatter(vbuf, [hi_pos], v_hi, mask=mlo4)
        plsc.store_scatter(fbuf, [r * TOPK + iota], f_lo)
        plsc.store_scatter(fbuf, [hi_pos], f_hi, mask=mlo4)
        kg_lo = b * K + lax.shift_right_logical(f_lo, 6)
        kg_hi = b * K + lax.shift_right_logical(f_hi, 6)
        ag_lo = b * A + (f_lo & 63)
        ag_hi = b * A + (f_hi & 63)
        plsc.store_scatter(izbuf, [r * TOPK + iota], kg_lo)
        plsc.store_scatter(izbuf, [hi_pos], kg_hi, mask=mlo4)
        plsc.store_scatter(iabuf, [r * TOPK + iota], ag_lo)
        plsc.store_scatter(iabuf, [hi_pos], ag_hi, mask=mlo4)

    cp1 = pltpu.async_copy(z_hbm.at[izbuf], zbufs, semg)
    cp2 = pltpu.async_copy(d_hbm.at[iabuf], dbufs, semg)
    cp1.wait()
    cp2.wait()
    pltpu.sync_copy(zbufs, zg_hbm.at[pl.ds(wid * nrow, nrow)])
    pltpu.sync_copy(dbufs, dg_hbm.at[pl.ds(wid * nrow, nrow)])
    pltpu.sync_copy(vbuf, val_hbm.at[pl.ds(wid * nrow, nrow)])
    pltpu.sync_copy(fbuf, idx_hbm.at[pl.ds(wid * nrow, nrow)])


def _tc_finish_body(zg_ref, dg_ref, val_ref, zout_ref, att_ref):
    zn = zg_ref[...] + dg_ref[...]
    mu = jnp.mean(zn, axis=2, keepdims=True)
    var = jnp.mean((zn - mu) ** 2, axis=2, keepdims=True)
    zout_ref[...] = (zn - mu) / jnp.sqrt(var + 1e-5)
    v = val_ref[...]
    att_ref[...] = v - v[:, 0:1]


@jax.jit
def _run(z, achs, ret2d, Wqd, Wk, bd2d):
    grid = (B // BB,)
    y3, disp = pl.pallas_call(
        _tc_scores_body,
        grid=grid,
        in_specs=[
            pl.BlockSpec((BB, K, D), lambda i: (i, 0, 0)),
            pl.BlockSpec((BB, A, D), lambda i: (i, 0, 0)),
            pl.BlockSpec((BB, K), lambda i: (i, 0)),
            pl.BlockSpec((D, 2 * D), lambda i: (0, 0)),
            pl.BlockSpec((D, D), lambda i: (0, 0)),
            pl.BlockSpec((1, D), lambda i: (0, 0)),
        ],
        out_specs=(
            pl.BlockSpec((BB, K, A), lambda i: (i, 0, 0)),
            pl.BlockSpec((BB, A, D), lambda i: (i, 0, 0)),
        ),
        out_shape=(
            jax.ShapeDtypeStruct((B, K, A), jnp.float32),
            jax.ShapeDtypeStruct((B, A, D), jnp.float32),
        ),
    )(z, achs, ret2d, Wqd, Wk, bd2d)

    sc = functools.partial(
        pl.kernel,
        out_type=(
            jax.ShapeDtypeStruct((B * TOPK, D), jnp.float32),
            jax.ShapeDtypeStruct((B * TOPK, D), jnp.float32),
            jax.ShapeDtypeStruct((B * TOPK,), jnp.float32),
            jax.ShapeDtypeStruct((B * TOPK,), jnp.int32),
        ),
        mesh=plsc.VectorSubcoreMesh(core_axis_name="c",
                                    subcore_axis_name="s"),
        compiler_params=pltpu.CompilerParams(needs_layout_passes=False),
        scratch_types=[
            pltpu.VMEM((_RPW * K * A,), jnp.float32),
            pltpu.VMEM((_RPW * TOPK,), jnp.int32),
            pltpu.VMEM((_RPW * TOPK,), jnp.int32),
            pltpu.VMEM((_RPW * TOPK, D), jnp.float32),
            pltpu.VMEM((_RPW * TOPK, D), jnp.float32),
            pltpu.VMEM((_RPW * TOPK,), jnp.float32),
            pltpu.VMEM((_RPW * TOPK,), jnp.int32),
            pltpu.SemaphoreType.DMA,
        ],
    )(_sc_topk_gather)
    zg, dg, vals, fids = sc(y3.reshape(B * K * A),
                            z.reshape(B * K, D),
                            disp.reshape(B * A, D))

    z_out, att = pl.pallas_call(
        _tc_finish_body,
        grid=grid,
        in_specs=[
            pl.BlockSpec((BB, TOPK, D), lambda i: (i, 0, 0)),
            pl.BlockSpec((BB, TOPK, D), lambda i: (i, 0, 0)),
            pl.BlockSpec((BB, TOPK), lambda i: (i, 0)),
        ],
        out_specs=(
            pl.BlockSpec((BB, TOPK, D), lambda i: (i, 0, 0)),
            pl.BlockSpec((BB, TOPK), lambda i: (i, 0)),
        ),
        out_shape=(
            jax.ShapeDtypeStruct((B, TOPK, D), jnp.float32),
            jax.ShapeDtypeStruct((B, TOPK), jnp.float32),
        ),
    )(zg.reshape(B, TOPK, D), dg.reshape(B, TOPK, D),
      vals.reshape(B, TOPK))
    idx = fids.reshape(B, TOPK)
    return z_out, att, idx


def kernel(z, achs, anchor_att_ret, Wq, Wk, Wd, bd):
    ret2d = anchor_att_ret.reshape(B, K)
    bd2d = bd.reshape(1, D)
    Wqd = jnp.concatenate([Wq, Wd], axis=1)
    z_out, att, idx = _run(z, achs, ret2d, Wqd, Wk, bd2d)
    return (z_out, att.reshape(B, TOPK, 1), idx)
